# baseline (speedup 1.0000x reference)
import functools

import jax
import jax.numpy as jnp
from jax import lax
from jax.experimental import pallas as pl
from jax.experimental.pallas import tpu as pltpu
from jax.experimental.pallas import tpu_sc as plsc

V = 100000
D = 128
CTX = 16384
NW = 32
ROWS_PER_W = CTX // NW
CHUNK = 128
NCHUNK = ROWS_PER_W // CHUNK
LANES = 16
NVEC = D // LANES

BLK = 20480
S_SC = BLK
T_TC = V - S_SC
NB_TC = 4

MV_RPW = S_SC // NW
MV_CH = 320
MV_NCH = MV_RPW // MV_CH
MV_NBUF = 2

NSUB = 16
GCH = CTX // NSUB // CHUNK
GBUF = 2


def _sc_gather_sum(idx2d, table):
    mesh = plsc.VectorSubcoreMesh(core_axis_name="c", subcore_axis_name="s")

    @functools.partial(
        pl.kernel,
        out_type=jax.ShapeDtypeStruct((NW, D), jnp.float32),
        mesh=mesh,
        scratch_types=[
            pltpu.VMEM((NCHUNK, CHUNK), jnp.int32),
            pltpu.VMEM((NCHUNK, CHUNK, D), jnp.float32),
            pltpu.VMEM((D,), jnp.float32),
            pltpu.SemaphoreType.DMA,
        ],
    )
    def k(idx_hbm, table_hbm, out_hbm, idx_v, rows_v, acc_v, sem):
        wid = lax.axis_index("s") * 2 + lax.axis_index("c")
        pltpu.sync_copy(idx_hbm.at[pl.ds(wid * NCHUNK, NCHUNK)], idx_v)
        copies = [
            pltpu.async_copy(table_hbm.at[idx_v.at[c]], rows_v.at[c], sem)
            for c in range(NCHUNK)
        ]

        def outer(acc, c):
            def body(i, acc):
                return tuple(
                    acc[j] + rows_v[c, i, pl.ds(j * LANES, LANES)]
                    for j in range(NVEC)
                )
            return lax.fori_loop(0, CHUNK, body, acc, unroll=4)

        acc = tuple(jnp.zeros((LANES,), jnp.float32) for _ in range(NVEC))
        for c in range(NCHUNK):
            copies[c].wait()
            acc = outer(acc, c)
        for j in range(NVEC):
            acc_v[pl.ds(j * LANES, LANES)] = acc[j]
        pltpu.sync_copy(acc_v, out_hbm.at[wid])

    return k(idx2d, table)


def _sc_matvec_pp(idx2d, table, W):
    mesh = plsc.VectorSubcoreMesh(core_axis_name="c", subcore_axis_name="s")

    @functools.partial(
        pl.kernel,
        out_type=jax.ShapeDtypeStruct((S_SC * LANES,), jnp.float32),
        mesh=mesh,
        scratch_types=[
            pltpu.VMEM((GCH, CHUNK), jnp.int32),
            pltpu.VMEM((GBUF, CHUNK, D), jnp.float32),
            pltpu.VMEM((D,), jnp.float32),
            pltpu.VMEM((NSUB, D), jnp.float32),
            pltpu.VMEM_SHARED((NSUB, D), jnp.float32),
            pltpu.VMEM((MV_NBUF, MV_CH, D), jnp.float32),
            pltpu.VMEM((MV_RPW * LANES,), jnp.float32),
            pltpu.SemaphoreType.DMA,
            pltpu.SemaphoreType.DMA,
            pltpu.SemaphoreType.DMA,
        ],
    )
    def k(idx_hbm, table_hbm, w_hbm, out_hbm, idx_v, rows_v, acc_v, part_v,
          shared, wbuf, pp_v, gsem, sem0, sem1):
        sid = lax.axis_index("s")
        wid = sid * 2 + lax.axis_index("c")
        base = wid * MV_RPW
        sems = [sem0, sem1]

        pltpu.sync_copy(idx_hbm.at[pl.ds(sid * GCH, GCH)], idx_v)
        gcopies = [None] * GCH
        for c in range(GBUF):
            gcopies[c] = pltpu.async_copy(
                table_hbm.at[idx_v.at[c]], rows_v.at[c], gsem)
        wcopies = [None] * MV_NCH
        for bi in range(MV_NBUF):
            wcopies[bi] = pltpu.async_copy(
                w_hbm.at[pl.ds(base + bi * MV_CH, MV_CH), :], wbuf.at[bi],
                sems[bi],
            )

        def outer(acc, c):
            def body(i, acc):
                return tuple(
                    acc[j] + rows_v[c % GBUF, i, pl.ds(j * LANES, LANES)]
                    for j in range(NVEC)
                )
            return lax.fori_loop(0, CHUNK, body, acc, unroll=4)

        acc = tuple(jnp.zeros((LANES,), jnp.float32) for _ in range(NVEC))
        for c in range(GCH):
            gcopies[c].wait()
            acc = outer(acc, c)
            if c + GBUF < GCH:
                gcopies[c + GBUF] = pltpu.async_copy(
                    table_hbm.at[idx_v.at[c + GBUF]],
                    rows_v.at[(c + GBUF) % GBUF], gsem)
        for j in range(NVEC):
            acc_v[pl.ds(j * LANES, LANES)] = acc[j]

        pltpu.sync_copy(acc_v, shared.at[sid])
        plsc.subcore_barrier()
        pltpu.sync_copy(shared, part_v)

        def red(i, acc):
            return tuple(
                acc[j] + part_v[i, pl.ds(j * LANES, LANES)]
                for j in range(NVEC)
            )
        s = lax.fori_loop(
            0, NSUB, red,
            tuple(jnp.zeros((LANES,), jnp.float32) for _ in range(NVEC)),
            unroll=4,
        )

        for c in range(MV_NCH):
            bi = c % MV_NBUF
            wcopies[c].wait()

            def row(i, _):
                p = [
                    wbuf[bi, i, pl.ds(j * LANES, LANES)] * s[j]
                    for j in range(NVEC)
                ]
                while len(p) > 1:
                    p = [p[k] + p[k + 1] for k in range(0, len(p) - 1, 2)] + (
                        [p[-1]] if len(p) % 2 else [])
                pp_v[pl.ds((c * MV_CH + i) * LANES, LANES)] = p[0]
                return 0
            lax.fori_loop(0, MV_CH, row, 0, unroll=4)

            if c + MV_NBUF < MV_NCH:
                wcopies[c + MV_NBUF] = pltpu.async_copy(
                    w_hbm.at[pl.ds(base + (c + MV_NBUF) * MV_CH, MV_CH), :],
                    wbuf.at[bi], sems[bi],
                )

        pltpu.sync_copy(pp_v, out_hbm.at[pl.ds(base * LANES, MV_RPW * LANES)])

    return k(idx2d, table, W)


def _tc_passthrough(partials):
    def body(p_ref, o_ref):
        o_ref[...] = p_ref[...]

    return pl.pallas_call(
        body,
        in_specs=[pl.BlockSpec(memory_space=pltpu.VMEM)],
        out_specs=pl.BlockSpec(memory_space=pltpu.VMEM),
        out_shape=jax.ShapeDtypeStruct((NW, D), jnp.float32),
    )(partials)


def _tc_logits_tail(partials, W, b2d):
    def body(part_ref, w_ref, b_ref, out_ref, m_ref, z_ref, m_s, s_s):
        j = pl.program_id(0)

        @pl.when(j == 0)
        def _():
            m_s[0] = -1e30
            s_s[0] = 0.0

        s = jnp.sum(part_ref[...], axis=0, keepdims=True)
        logits = lax.dot_general(
            s, w_ref[...], (((1,), (1,)), ((), ())),
            preferred_element_type=jnp.float32,
        ) + b_ref[...]
        col = S_SC + j * BLK + lax.broadcasted_iota(jnp.int32, (1, BLK), 1)
        logits = jnp.where(col < V, logits, -1e30)
        out_ref[...] = logits

        m_old = m_s[0]
        s_old = s_s[0]
        m_new = jnp.maximum(m_old, jnp.max(logits))
        s_new = s_old * jnp.exp(m_old - m_new) + jnp.sum(jnp.exp(logits - m_new))
        m_s[0] = m_new
        s_s[0] = s_new

        @pl.when(j == NB_TC - 1)
        def _():
            m_ref[0, 0] = m_new
            z_ref[0, 0] = s_new

    return pl.pallas_call(
        body,
        grid=(NB_TC,),
        in_specs=[
            pl.BlockSpec((NW, D), lambda j: (0, 0)),
            pl.BlockSpec((BLK, D), lambda j: (j + S_SC // BLK, 0)),
            pl.BlockSpec((1, BLK), lambda j: (0, j + S_SC // BLK)),
        ],
        out_specs=[
            pl.BlockSpec((1, BLK), lambda j: (0, j)),
            pl.BlockSpec((1, 1), lambda j: (0, 0), memory_space=pltpu.SMEM),
            pl.BlockSpec((1, 1), lambda j: (0, 0), memory_space=pltpu.SMEM),
        ],
        out_shape=[
            jax.ShapeDtypeStruct((1, NB_TC * BLK), jnp.float32),
            jax.ShapeDtypeStruct((1, 1), jnp.float32),
            jax.ShapeDtypeStruct((1, 1), jnp.float32),
        ],
        scratch_shapes=[
            pltpu.SMEM((1,), jnp.float32),
            pltpu.SMEM((1,), jnp.float32),
        ],
    )(partials, W, b2d)


def _tc_merge(ppq, b_sc2d, logits_tc, m_tc, z_tc):
    Q = S_SC // 128

    def body(pp_ref, b_ref, ltc_ref, m_ref, z_ref, osc_ref, otc_ref):
        fold = (lax.broadcasted_iota(jnp.int32, (16 * 128, 128), 0) // LANES
                == lax.broadcasted_iota(jnp.int32, (16 * 128, 128), 1)
                ).astype(jnp.float32)
        lsc = lax.dot_general(
            pp_ref[...], fold, (((1,), (0,)), ((), ())),
            preferred_element_type=jnp.float32,
            precision=lax.Precision.HIGHEST,
        ) + b_ref[...]
        m_sc = jnp.max(lsc)
        z_sc = jnp.sum(jnp.exp(lsc - m_sc))
        m_t = m_ref[0, 0]
        z_t = z_ref[0, 0]
        m = jnp.maximum(m_sc, m_t)
        z = z_sc * jnp.exp(m_sc - m) + z_t * jnp.exp(m_t - m)
        lse = m + jnp.log(z)
        osc_ref[...] = lsc - lse
        otc_ref[...] = ltc_ref[...] - lse

    return pl.pallas_call(
        body,
        in_specs=[
            pl.BlockSpec(memory_space=pltpu.VMEM),
            pl.BlockSpec(memory_space=pltpu.VMEM),
            pl.BlockSpec(memory_space=pltpu.VMEM),
            pl.BlockSpec(memory_space=pltpu.SMEM),
            pl.BlockSpec(memory_space=pltpu.SMEM),
        ],
        out_specs=[
            pl.BlockSpec(memory_space=pltpu.VMEM),
            pl.BlockSpec(memory_space=pltpu.VMEM),
        ],
        out_shape=[
            jax.ShapeDtypeStruct((Q, 128), jnp.float32),
            jax.ShapeDtypeStruct((1, NB_TC * BLK), jnp.float32),
        ],
    )(ppq, b_sc2d, logits_tc, m_tc, z_tc)


def kernel(inputs, emb_table, W, b):
    idx2d = inputs.astype(jnp.int32).reshape(NW * NCHUNK, CHUNK)
    partials = _sc_gather_sum(idx2d, emb_table)
    pp = _sc_matvec_pp(idx2d, emb_table, W)
    logits_tc, m_tc, z_tc = _tc_logits_tail(partials, W, b.reshape(1, V))
    out_sc, out_tc = _tc_merge(
        pp.reshape(S_SC // 128, 16 * 128),
        b[:S_SC].reshape(S_SC // 128, 128),
        logits_tc, m_tc, z_tc,
    )
    return jnp.concatenate(
        [out_sc.reshape(1, S_SC), out_tc[:, :T_TC]], axis=1
    )


# speedup vs baseline: 1.6239x; 1.6239x over previous
import functools

import jax
import jax.numpy as jnp
from jax import lax
from jax.experimental import pallas as pl
from jax.experimental.pallas import tpu as pltpu
from jax.experimental.pallas import tpu_sc as plsc

V = 100000
D = 128
CTX = 16384
NW = 32
ROWS_PER_W = CTX // NW
CHUNK = 128
NCHUNK = ROWS_PER_W // CHUNK
LANES = 16
NVEC = D // LANES

BLK = 20480
S_SC = 0
T_TC = V - S_SC
NB_TC = 5

MV_RPW = S_SC // NW
MV_CH = 320
MV_NCH = MV_RPW // MV_CH
MV_NBUF = 2

NSUB = 16
GCH = CTX // NSUB // CHUNK
GBUF = 2


def _sc_gather_sum(idx2d, table):
    mesh = plsc.VectorSubcoreMesh(core_axis_name="c", subcore_axis_name="s")

    @functools.partial(
        pl.kernel,
        out_type=jax.ShapeDtypeStruct((NW, D), jnp.float32),
        mesh=mesh,
        scratch_types=[
            pltpu.VMEM((NCHUNK, CHUNK), jnp.int32),
            pltpu.VMEM((NCHUNK, CHUNK, D), jnp.float32),
            pltpu.VMEM((D,), jnp.float32),
            pltpu.SemaphoreType.DMA,
        ],
    )
    def k(idx_hbm, table_hbm, out_hbm, idx_v, rows_v, acc_v, sem):
        wid = lax.axis_index("s") * 2 + lax.axis_index("c")
        pltpu.sync_copy(idx_hbm.at[pl.ds(wid * NCHUNK, NCHUNK)], idx_v)
        copies = [
            pltpu.async_copy(table_hbm.at[idx_v.at[c]], rows_v.at[c], sem)
            for c in range(NCHUNK)
        ]

        def outer(acc, c):
            def body(i, acc):
                return tuple(
                    acc[j] + rows_v[c, i, pl.ds(j * LANES, LANES)]
                    for j in range(NVEC)
                )
            return lax.fori_loop(0, CHUNK, body, acc, unroll=4)

        acc = tuple(jnp.zeros((LANES,), jnp.float32) for _ in range(NVEC))
        for c in range(NCHUNK):
            copies[c].wait()
            acc = outer(acc, c)
        for j in range(NVEC):
            acc_v[pl.ds(j * LANES, LANES)] = acc[j]
        pltpu.sync_copy(acc_v, out_hbm.at[wid])

    return k(idx2d, table)


def _sc_matvec_pp(idx2d, table, W):
    mesh = plsc.VectorSubcoreMesh(core_axis_name="c", subcore_axis_name="s")

    @functools.partial(
        pl.kernel,
        out_type=jax.ShapeDtypeStruct((S_SC * LANES,), jnp.float32),
        mesh=mesh,
        scratch_types=[
            pltpu.VMEM((GCH, CHUNK), jnp.int32),
            pltpu.VMEM((GBUF, CHUNK, D), jnp.float32),
            pltpu.VMEM((D,), jnp.float32),
            pltpu.VMEM((NSUB, D), jnp.float32),
            pltpu.VMEM_SHARED((NSUB, D), jnp.float32),
            pltpu.VMEM((MV_NBUF, MV_CH, D), jnp.float32),
            pltpu.VMEM((MV_RPW * LANES,), jnp.float32),
            pltpu.SemaphoreType.DMA,
            pltpu.SemaphoreType.DMA,
            pltpu.SemaphoreType.DMA,
        ],
    )
    def k(idx_hbm, table_hbm, w_hbm, out_hbm, idx_v, rows_v, acc_v, part_v,
          shared, wbuf, pp_v, gsem, sem0, sem1):
        sid = lax.axis_index("s")
        wid = sid * 2 + lax.axis_index("c")
        base = wid * MV_RPW
        sems = [sem0, sem1]

        pltpu.sync_copy(idx_hbm.at[pl.ds(sid * GCH, GCH)], idx_v)
        gcopies = [None] * GCH
        for c in range(GBUF):
            gcopies[c] = pltpu.async_copy(
                table_hbm.at[idx_v.at[c]], rows_v.at[c], gsem)
        wcopies = [None] * MV_NCH
        for bi in range(MV_NBUF):
            wcopies[bi] = pltpu.async_copy(
                w_hbm.at[pl.ds(base + bi * MV_CH, MV_CH), :], wbuf.at[bi],
                sems[bi],
            )

        def outer(acc, c):
            def body(i, acc):
                return tuple(
                    acc[j] + rows_v[c % GBUF, i, pl.ds(j * LANES, LANES)]
                    for j in range(NVEC)
                )
            return lax.fori_loop(0, CHUNK, body, acc, unroll=4)

        acc = tuple(jnp.zeros((LANES,), jnp.float32) for _ in range(NVEC))
        for c in range(GCH):
            gcopies[c].wait()
            acc = outer(acc, c)
            if c + GBUF < GCH:
                gcopies[c + GBUF] = pltpu.async_copy(
                    table_hbm.at[idx_v.at[c + GBUF]],
                    rows_v.at[(c + GBUF) % GBUF], gsem)
        for j in range(NVEC):
            acc_v[pl.ds(j * LANES, LANES)] = acc[j]

        pltpu.sync_copy(acc_v, shared.at[sid])
        plsc.subcore_barrier()
        pltpu.sync_copy(shared, part_v)

        def red(i, acc):
            return tuple(
                acc[j] + part_v[i, pl.ds(j * LANES, LANES)]
                for j in range(NVEC)
            )
        s = lax.fori_loop(
            0, NSUB, red,
            tuple(jnp.zeros((LANES,), jnp.float32) for _ in range(NVEC)),
            unroll=4,
        )

        for c in range(MV_NCH):
            bi = c % MV_NBUF
            wcopies[c].wait()

            def row(i, _):
                p = [
                    wbuf[bi, i, pl.ds(j * LANES, LANES)] * s[j]
                    for j in range(NVEC)
                ]
                while len(p) > 1:
                    p = [p[k] + p[k + 1] for k in range(0, len(p) - 1, 2)] + (
                        [p[-1]] if len(p) % 2 else [])
                pp_v[pl.ds((c * MV_CH + i) * LANES, LANES)] = p[0]
                return 0
            lax.fori_loop(0, MV_CH, row, 0, unroll=4)

            if c + MV_NBUF < MV_NCH:
                wcopies[c + MV_NBUF] = pltpu.async_copy(
                    w_hbm.at[pl.ds(base + (c + MV_NBUF) * MV_CH, MV_CH), :],
                    wbuf.at[bi], sems[bi],
                )

        pltpu.sync_copy(pp_v, out_hbm.at[pl.ds(base * LANES, MV_RPW * LANES)])

    return k(idx2d, table, W)


def _tc_passthrough(partials):
    def body(p_ref, o_ref):
        o_ref[...] = p_ref[...]

    return pl.pallas_call(
        body,
        in_specs=[pl.BlockSpec(memory_space=pltpu.VMEM)],
        out_specs=pl.BlockSpec(memory_space=pltpu.VMEM),
        out_shape=jax.ShapeDtypeStruct((NW, D), jnp.float32),
    )(partials)


def _tc_logits_tail(partials, W, b2d):
    def body(part_ref, w_ref, b_ref, out_ref, m_ref, z_ref, m_s, s_s):
        j = pl.program_id(0)

        @pl.when(j == 0)
        def _():
            m_s[0] = -1e30
            s_s[0] = 0.0

        s = jnp.sum(part_ref[...], axis=0, keepdims=True)
        logits = lax.dot_general(
            s, w_ref[...], (((1,), (1,)), ((), ())),
            preferred_element_type=jnp.float32,
        ) + b_ref[...]
        col = S_SC + j * BLK + lax.broadcasted_iota(jnp.int32, (1, BLK), 1)
        logits = jnp.where(col < V, logits, -1e30)
        out_ref[...] = logits

        m_old = m_s[0]
        s_old = s_s[0]
        m_new = jnp.maximum(m_old, jnp.max(logits))
        s_new = s_old * jnp.exp(m_old - m_new) + jnp.sum(jnp.exp(logits - m_new))
        m_s[0] = m_new
        s_s[0] = s_new

        @pl.when(j == NB_TC - 1)
        def _():
            m_ref[0, 0] = m_new
            z_ref[0, 0] = s_new

    return pl.pallas_call(
        body,
        grid=(NB_TC,),
        in_specs=[
            pl.BlockSpec((NW, D), lambda j: (0, 0)),
            pl.BlockSpec((BLK, D), lambda j: (j + S_SC // BLK, 0)),
            pl.BlockSpec((1, BLK), lambda j: (0, j + S_SC // BLK)),
        ],
        out_specs=[
            pl.BlockSpec((1, BLK), lambda j: (0, j)),
            pl.BlockSpec((1, 1), lambda j: (0, 0), memory_space=pltpu.SMEM),
            pl.BlockSpec((1, 1), lambda j: (0, 0), memory_space=pltpu.SMEM),
        ],
        out_shape=[
            jax.ShapeDtypeStruct((1, NB_TC * BLK), jnp.float32),
            jax.ShapeDtypeStruct((1, 1), jnp.float32),
            jax.ShapeDtypeStruct((1, 1), jnp.float32),
        ],
        scratch_shapes=[
            pltpu.SMEM((1,), jnp.float32),
            pltpu.SMEM((1,), jnp.float32),
        ],
    )(partials, W, b2d)


def _tc_merge(ppq, b_sc2d, logits_tc, m_tc, z_tc):
    Q = S_SC // 128

    def body(pp_ref, b_ref, ltc_ref, m_ref, z_ref, osc_ref, otc_ref):
        fold = (lax.broadcasted_iota(jnp.int32, (16 * 128, 128), 0) // LANES
                == lax.broadcasted_iota(jnp.int32, (16 * 128, 128), 1)
                ).astype(jnp.float32)
        lsc = lax.dot_general(
            pp_ref[...], fold, (((1,), (0,)), ((), ())),
            preferred_element_type=jnp.float32,
            precision=lax.Precision.HIGHEST,
        ) + b_ref[...]
        m_sc = jnp.max(lsc)
        z_sc = jnp.sum(jnp.exp(lsc - m_sc))
        m_t = m_ref[0, 0]
        z_t = z_ref[0, 0]
        m = jnp.maximum(m_sc, m_t)
        z = z_sc * jnp.exp(m_sc - m) + z_t * jnp.exp(m_t - m)
        lse = m + jnp.log(z)
        osc_ref[...] = lsc - lse
        otc_ref[...] = ltc_ref[...] - lse

    return pl.pallas_call(
        body,
        in_specs=[
            pl.BlockSpec(memory_space=pltpu.VMEM),
            pl.BlockSpec(memory_space=pltpu.VMEM),
            pl.BlockSpec(memory_space=pltpu.VMEM),
            pl.BlockSpec(memory_space=pltpu.SMEM),
            pl.BlockSpec(memory_space=pltpu.SMEM),
        ],
        out_specs=[
            pl.BlockSpec(memory_space=pltpu.VMEM),
            pl.BlockSpec(memory_space=pltpu.VMEM),
        ],
        out_shape=[
            jax.ShapeDtypeStruct((Q, 128), jnp.float32),
            jax.ShapeDtypeStruct((1, NB_TC * BLK), jnp.float32),
        ],
    )(ppq, b_sc2d, logits_tc, m_tc, z_tc)


def _tc_normalize(logits, m, z):
    def body(l_ref, m_ref, z_ref, o_ref):
        lse = m_ref[0, 0] + jnp.log(z_ref[0, 0])
        o_ref[...] = l_ref[:, :V] - lse

    return pl.pallas_call(
        body,
        in_specs=[
            pl.BlockSpec(memory_space=pltpu.VMEM),
            pl.BlockSpec(memory_space=pltpu.SMEM),
            pl.BlockSpec(memory_space=pltpu.SMEM),
        ],
        out_specs=pl.BlockSpec(memory_space=pltpu.VMEM),
        out_shape=jax.ShapeDtypeStruct((1, V), jnp.float32),
    )(logits, m, z)


def kernel(inputs, emb_table, W, b):
    idx2d = inputs.astype(jnp.int32).reshape(NW * NCHUNK, CHUNK)
    partials = _sc_gather_sum(idx2d, emb_table)
    logits, m, z = _tc_logits_tail(partials, W, b.reshape(1, V))
    return _tc_normalize(logits, m, z)


# speedup vs baseline: 1.7011x; 1.0476x over previous
import functools

import jax
import jax.numpy as jnp
from jax import lax
from jax.experimental import pallas as pl
from jax.experimental.pallas import tpu as pltpu
from jax.experimental.pallas import tpu_sc as plsc

V = 100000
D = 128
CTX = 16384
NW = 32
ROWS_PER_W = CTX // NW
CHUNK = 128
NCHUNK = ROWS_PER_W // CHUNK
LANES = 16
NVEC = D // LANES

BLK = 25600
NB = (V + BLK - 1) // BLK


def _sc_gather_sum(idx2d, table):
    mesh = plsc.VectorSubcoreMesh(core_axis_name="c", subcore_axis_name="s")

    @functools.partial(
        pl.kernel,
        out_type=jax.ShapeDtypeStruct((NW, D), jnp.float32),
        mesh=mesh,
        scratch_types=[
            pltpu.VMEM((NCHUNK, CHUNK), jnp.int32),
            pltpu.VMEM((NCHUNK, CHUNK, D), jnp.float32),
            pltpu.VMEM((D,), jnp.float32),
            pltpu.SemaphoreType.DMA,
        ],
    )
    def k(idx_hbm, table_hbm, out_hbm, idx_v, rows_v, acc_v, sem):
        wid = lax.axis_index("s") * 2 + lax.axis_index("c")
        pltpu.sync_copy(idx_hbm.at[pl.ds(wid * NCHUNK, NCHUNK)], idx_v)
        copies = [
            pltpu.async_copy(table_hbm.at[idx_v.at[c]], rows_v.at[c], sem)
            for c in range(NCHUNK)
        ]

        def outer(acc, c):
            def body(i, acc):
                return tuple(
                    acc[j] + rows_v[c, i, pl.ds(j * LANES, LANES)]
                    for j in range(NVEC)
                )
            return lax.fori_loop(0, CHUNK, body, acc, unroll=4)

        acc = tuple(jnp.zeros((LANES,), jnp.float32) for _ in range(NVEC))
        for c in range(NCHUNK):
            copies[c].wait()
            acc = outer(acc, c)
        for j in range(NVEC):
            acc_v[pl.ds(j * LANES, LANES)] = acc[j]
        pltpu.sync_copy(acc_v, out_hbm.at[wid])

    return k(idx2d, table)


def _tc_log_probs(partials, W, b2d):
    def body(part_ref, w_ref, b_ref, out_ref, log_v, m_s, s_s):
        j = pl.program_id(0)

        @pl.when(j == 0)
        def _():
            m_s[0] = -1e30
            s_s[0] = 0.0

        s = jnp.sum(part_ref[...], axis=0, keepdims=True)
        logits = lax.dot_general(
            s, w_ref[...], (((1,), (1,)), ((), ())),
            preferred_element_type=jnp.float32,
        ) + b_ref[...]
        col = j * BLK + lax.broadcasted_iota(jnp.int32, (1, BLK), 1)
        logits = jnp.where(col < V, logits, -1e30)
        log_v[j] = logits

        m_old = m_s[0]
        s_old = s_s[0]
        m_new = jnp.maximum(m_old, jnp.max(logits))
        s_new = s_old * jnp.exp(m_old - m_new) + jnp.sum(jnp.exp(logits - m_new))
        m_s[0] = m_new
        s_s[0] = s_new

        @pl.when(j == NB - 1)
        def _():
            lse = m_new + jnp.log(s_new)
            for k in range(NB):
                width = min(BLK, V - k * BLK)
                out_ref[:, k * BLK:k * BLK + width] = (
                    log_v[k][:, :width] - lse
                )

    return pl.pallas_call(
        body,
        grid=(NB,),
        in_specs=[
            pl.BlockSpec((NW, D), lambda j: (0, 0)),
            pl.BlockSpec((BLK, D), lambda j: (j, 0)),
            pl.BlockSpec((1, BLK), lambda j: (0, j)),
        ],
        out_specs=pl.BlockSpec((1, V), lambda j: (0, 0)),
        out_shape=jax.ShapeDtypeStruct((1, V), jnp.float32),
        scratch_shapes=[
            pltpu.VMEM((NB, 1, BLK), jnp.float32),
            pltpu.SMEM((1,), jnp.float32),
            pltpu.SMEM((1,), jnp.float32),
        ],
    )(partials, W, b2d)


def kernel(inputs, emb_table, W, b):
    idx2d = inputs.astype(jnp.int32).reshape(NW * NCHUNK, CHUNK)
    partials = _sc_gather_sum(idx2d, emb_table)
    return _tc_log_probs(partials, W, b.reshape(1, V))
